# baseline (speedup 1.0000x reference)
import jax
import jax.numpy as jnp
from jax.experimental import pallas as pl

EPS = 1e-5


def _gcn(x, W, b, src, dst):
    n = x.shape[0]
    loop = jnp.arange(n)
    s = jnp.concatenate([src, loop])
    d = jnp.concatenate([dst, loop])
    deg = jnp.zeros((n,), x.dtype).at[d].add(1.0)
    dinv = 1.0 / jnp.sqrt(deg)
    norm = dinv[s] * dinv[d]
    h = x @ W
    out = jnp.zeros((n, W.shape[1]), x.dtype).at[d].add(h[s] * norm[:, None])
    return out + b


def _bn(x, g, b):
    m = x.mean(0)
    v = x.var(0)
    return (x - m) / jnp.sqrt(v + EPS) * g + b


def _identity_kernel(x_ref, o_ref):
    o_ref[...] = x_ref[...]


def _pl_id(x):
    return pl.pallas_call(
        _identity_kernel,
        out_shape=jax.ShapeDtypeStruct(x.shape, x.dtype),
    )(x)


def kernel(x, edge_index,
           mu_W1, mu_b1, mu_W2, mu_b2, mu_W3, mu_b3, mu_g1, mu_be1, mu_g2, mu_be2,
           log_W1, log_b1, log_W2, log_b2, log_W3, log_b3, log_g1, log_be1, log_g2, log_be2):
    src, dst = edge_index[0], edge_index[1]
    h = _gcn(x, mu_W1, mu_b1, src, dst)
    h = _bn(h, mu_g1, mu_be1)
    h = jax.nn.relu(h)
    h = _gcn(h, mu_W2, mu_b2, src, dst)
    h = _bn(h, mu_g2, mu_be2)
    h = jax.nn.relu(h)
    mu = _gcn(h, mu_W3, mu_b3, src, dst)

    g = _gcn(x, log_W1, log_b1, src, dst)
    g = jax.nn.relu(g)
    g = _bn(g, log_g1, log_be1)
    g = _gcn(g, log_W2, log_b2, src, dst)
    g = jax.nn.relu(g)
    g = _bn(g, log_g2, log_be2)
    log = _gcn(g, log_W3, log_b3, src, dst)
    return (_pl_id(mu), _pl_id(log))


# speedup vs baseline: 7.0316x; 7.0316x over previous
import jax
import jax.numpy as jnp
from jax import lax
from jax.experimental import pallas as pl
from jax.experimental.pallas import tpu as pltpu
from jax.experimental.pallas import tpu_sc as plsc

EPS = 1e-5
NC = 2
NS = 16
NW = NC * NS
KE = 128
BROWS = 1000


def _make_spmm(n_acc, d, iters):
    rpt = n_acc // NS
    mesh = plsc.VectorSubcoreMesh(core_axis_name="c", subcore_axis_name="s")

    def body(zp, src3, dst3, zeros, out, src_v, dst_v, rows_v, acc, sem):
        cid = lax.axis_index("c")
        sid = lax.axis_index("s")
        wid = cid * NS + sid
        pltpu.sync_copy(zeros.at[pl.ds(sid * rpt, rpt)], acc.at[pl.ds(sid * rpt, rpt)])
        pltpu.sync_copy(src3.at[wid], src_v)
        pltpu.sync_copy(dst3.at[wid], dst_v)
        plsc.subcore_barrier()

        def step(i, c):
            pltpu.async_copy(zp.at[src_v.at[i]], rows_v, sem).wait()
            pltpu.sync_copy(rows_v, acc.at[dst_v.at[i]], add=True)
            return c

        lax.fori_loop(0, iters, step, 0)
        plsc.subcore_barrier()
        pltpu.sync_copy(acc.at[pl.ds(sid * rpt, rpt)],
                        out.at[cid, pl.ds(sid * rpt, rpt)])

    return pl.kernel(
        body,
        out_type=jax.ShapeDtypeStruct((NC, n_acc, d), jnp.float32),
        mesh=mesh,
        scratch_types=[
            pltpu.VMEM((iters, KE), jnp.int32),
            pltpu.VMEM((iters, KE), jnp.int32),
            pltpu.VMEM((KE, d), jnp.float32),
            pltpu.VMEM_SHARED((n_acc, d), jnp.float32),
            pltpu.SemaphoreType.DMA,
        ],
    )


def _dinv(dp):
    return 1.0 / jnp.sqrt(dp[0, :, 0:1] + dp[1, :, 0:1] + 1.0)


def _mm(mode, a_list, s_list, gbe_list, deg_p, w, dout, n):
    cin = len(a_list)
    nblk = n // BROWS
    dcs = [128] * (dout // 128) + ([dout % 128] if dout % 128 else [])

    in_specs = [pl.BlockSpec((2, BROWS, 128), lambda i: (0, i, 0))]
    args = [deg_p]
    for c in range(cin):
        in_specs.append(pl.BlockSpec((BROWS, 128), lambda i: (i, 0)))
        args.append(a_list[c])
        if mode != "id":
            in_specs.append(pl.BlockSpec((8, 128), lambda i: (0, 0)))
            args.append(s_list[c])
            in_specs.append(pl.BlockSpec((8, 128), lambda i: (0, 0)))
            args.append(gbe_list[c])
    in_specs.append(pl.BlockSpec(w.shape, lambda i: (0, 0)))
    args.append(w)
    out_specs = [pl.BlockSpec((BROWS, dc), lambda i: (i, 0)) for dc in dcs]
    out_shape = [jax.ShapeDtypeStruct((n, dc), jnp.float32) for dc in dcs]
    inv_n = 1.0 / n

    def body(*refs):
        dp = refs[0][...]
        dinv = _dinv(dp)
        k = 1
        us = []
        for _ in range(cin):
            a = refs[k][...]
            k += 1
            if mode == "id":
                u = a
            else:
                s = refs[k][...]
                k += 1
                gbe = refs[k][...]
                k += 1
                m = s[0] * inv_n
                var = s[1] * inv_n - m * m
                istd = 1.0 / jnp.sqrt(var + EPS)
                if mode == "mu":
                    u = jnp.maximum((a - m) * istd * gbe[0] + gbe[1], 0.0)
                else:
                    u = (jnp.maximum(a, 0.0) - m) * istd * gbe[0] + gbe[1]
            us.append(u)
        wv = refs[k][...]
        k += 1
        z = jnp.dot(us[0], wv[0:128], preferred_element_type=jnp.float32)
        for c in range(1, cin):
            z = z + jnp.dot(us[c], wv[128 * c:128 * (c + 1)],
                            preferred_element_type=jnp.float32)
        z = z * dinv
        off = 0
        for o, dc in zip(refs[k:], dcs):
            o[...] = z[:, off:off + dc]
            off += dc

    outs = pl.pallas_call(body, grid=(nblk,), in_specs=in_specs,
                          out_specs=out_specs, out_shape=out_shape)(*args)
    return outs


def _combine(mode, z, p, deg_p, bpk, n, p_col_off=0):
    d = z.shape[1]
    dp_full = p.shape[2]
    nblk = n // BROWS
    stats = mode != "none"
    out_shape = [jax.ShapeDtypeStruct((n, d), jnp.float32)]
    out_specs = [pl.BlockSpec((BROWS, d), lambda i: (i, 0))]
    if stats:
        out_shape.append(jax.ShapeDtypeStruct((8, d), jnp.float32))
        out_specs.append(pl.BlockSpec((8, d), lambda i: (0, 0)))

    def body(z_ref, p_ref, dp_ref, b_ref, a_ref, *s_refs):
        i = pl.program_id(0)
        dinv = _dinv(dp_ref[...])
        pv = p_ref[...][:, :, p_col_off:p_col_off + d]
        a = dinv * (pv[0] + pv[1] + z_ref[...]) + b_ref[...][0]
        a_ref[...] = a
        if stats:
            s_ref = s_refs[0]
            t = a if mode == "mu" else jnp.maximum(a, 0.0)
            upd = jnp.concatenate(
                [jnp.sum(t, axis=0, keepdims=True),
                 jnp.sum(t * t, axis=0, keepdims=True),
                 jnp.zeros((6, d), jnp.float32)], axis=0)

            @pl.when(i == 0)
            def _():
                s_ref[...] = upd

            @pl.when(i != 0)
            def _():
                s_ref[...] = s_ref[...] + upd

    in_specs = [
        pl.BlockSpec((BROWS, d), lambda i: (i, 0)),
        pl.BlockSpec((2, BROWS, dp_full), lambda i: (0, i, 0)),
        pl.BlockSpec((2, BROWS, 128), lambda i: (0, i, 0)),
        pl.BlockSpec((8, d), lambda i: (0, 0)),
    ]
    res = pl.pallas_call(body, grid=(nblk,), in_specs=in_specs,
                         out_specs=out_specs, out_shape=out_shape)(z, p, deg_p, bpk)
    return res if stats else res[0]


def _pk8(v):
    return jnp.zeros((8, v.shape[0]), jnp.float32).at[0].set(v)


def _gbe(g, be):
    return jnp.zeros((8, g.shape[0]), jnp.float32).at[0].set(g).at[1].set(be)


def kernel(x, edge_index,
           mu_W1, mu_b1, mu_W2, mu_b2, mu_W3, mu_b3, mu_g1, mu_be1, mu_g2, mu_be2,
           log_W1, log_b1, log_W2, log_b2, log_W3, log_b3, log_g1, log_be1, log_g2, log_be2):
    n = x.shape[0]
    e = edge_index.shape[1]
    src = edge_index[0].astype(jnp.int32)
    dst = edge_index[1].astype(jnp.int32)

    per = NW * KE
    e_pad = ((e + per - 1) // per) * per
    iters = e_pad // per
    pad = e_pad - e
    src3 = jnp.concatenate([src, jnp.zeros((pad,), jnp.int32)]).reshape(NW, iters, KE)
    dst3 = jnp.concatenate([dst, jnp.full((pad,), n, jnp.int32)]).reshape(NW, iters, KE)

    n_acc = ((n + 1 + 127) // 128) * 128
    spmm128 = _make_spmm(n_acc, 128, iters)
    zeros128 = jnp.zeros((n_acc, 128), jnp.float32)

    deg_p = spmm128(jnp.ones((n, 128), jnp.float32), src3, dst3, zeros128)

    w1 = jnp.concatenate([mu_W1, log_W1], axis=1)
    z_mu1, z_log1 = _mm("id", [x], None, None, deg_p, w1, 256, n)
    p_mu1 = spmm128(z_mu1, src3, dst3, zeros128)
    a_mu1, s_mu1 = _combine("mu", z_mu1, p_mu1, deg_p, _pk8(mu_b1), n)
    p_log1 = spmm128(z_log1, src3, dst3, zeros128)
    a_log1, s_log1 = _combine("log", z_log1, p_log1, deg_p, _pk8(log_b1), n)

    z_mu2a, z_mu2b = _mm("mu", [a_mu1], [s_mu1], [_gbe(mu_g1, mu_be1)],
                         deg_p, mu_W2, 256, n)
    z_log2a, z_log2b = _mm("log", [a_log1], [s_log1], [_gbe(log_g1, log_be1)],
                           deg_p, log_W2, 256, n)
    p_mu2a = spmm128(z_mu2a, src3, dst3, zeros128)
    a_mu2a, s_mu2a = _combine("mu", z_mu2a, p_mu2a, deg_p, _pk8(mu_b2[:128]), n)
    p_mu2b = spmm128(z_mu2b, src3, dst3, zeros128)
    a_mu2b, s_mu2b = _combine("mu", z_mu2b, p_mu2b, deg_p, _pk8(mu_b2[128:]), n)
    p_log2a = spmm128(z_log2a, src3, dst3, zeros128)
    a_log2a, s_log2a = _combine("log", z_log2a, p_log2a, deg_p, _pk8(log_b2[:128]), n)
    p_log2b = spmm128(z_log2b, src3, dst3, zeros128)
    a_log2b, s_log2b = _combine("log", z_log2b, p_log2b, deg_p, _pk8(log_b2[128:]), n)

    (z_mu3,) = _mm("mu", [a_mu2a, a_mu2b], [s_mu2a, s_mu2b],
                   [_gbe(mu_g2[:128], mu_be2[:128]), _gbe(mu_g2[128:], mu_be2[128:])],
                   deg_p, mu_W3, 64, n)
    (z_log3,) = _mm("log", [a_log2a, a_log2b], [s_log2a, s_log2b],
                    [_gbe(log_g2[:128], log_be2[:128]), _gbe(log_g2[128:], log_be2[128:])],
                    deg_p, log_W3, 64, n)
    z3 = jnp.concatenate([z_mu3, z_log3], axis=1)
    p3 = spmm128(z3, src3, dst3, zeros128)
    mu = _combine("none", z_mu3, p3, deg_p, _pk8(mu_b3), n, p_col_off=0)
    log = _combine("none", z_log3, p3, deg_p, _pk8(log_b3), n, p_col_off=64)
    return (mu, log)


# speedup vs baseline: 7.0517x; 1.0029x over previous
import jax
import jax.numpy as jnp
from jax import lax
from jax.experimental import pallas as pl
from jax.experimental.pallas import tpu as pltpu
from jax.experimental.pallas import tpu_sc as plsc

EPS = 1e-5
NC = 2
NS = 16
NW = NC * NS
KE = 128
BROWS = 1000


def _make_spmm(n_acc, d, iters):
    rpt = n_acc // NS
    nbuf = 4 if iters % 4 == 0 else (2 if iters % 2 == 0 else 1)
    mesh = plsc.VectorSubcoreMesh(core_axis_name="c", subcore_axis_name="s")

    def body(zp, src3, dst3, zeros, out, src_v, dst_v, rows_v, acc, *sems):
        cid = lax.axis_index("c")
        sid = lax.axis_index("s")
        wid = cid * NS + sid
        pltpu.sync_copy(zeros.at[pl.ds(sid * rpt, rpt)], acc.at[pl.ds(sid * rpt, rpt)])
        pltpu.sync_copy(src3.at[wid], src_v)
        pltpu.sync_copy(dst3.at[wid], dst_v)
        plsc.subcore_barrier()

        for b in range(nbuf):
            pltpu.async_copy(zp.at[src_v.at[b]], rows_v.at[b], sems[b])

        def step(i2, c):
            for b in range(nbuf):
                i = i2 * nbuf + b
                pltpu.make_async_copy(zp.at[src_v.at[i]], rows_v.at[b],
                                      sems[b]).wait()
                pltpu.sync_copy(rows_v.at[b], acc.at[dst_v.at[i]], add=True)
                nxt = i + nbuf

                @pl.when(nxt < iters)
                def _():
                    pltpu.async_copy(zp.at[src_v.at[nxt]], rows_v.at[b], sems[b])
            return c

        lax.fori_loop(0, iters // nbuf, step, 0)
        plsc.subcore_barrier()
        pltpu.sync_copy(acc.at[pl.ds(sid * rpt, rpt)],
                        out.at[cid, pl.ds(sid * rpt, rpt)])

    return pl.kernel(
        body,
        out_type=jax.ShapeDtypeStruct((NC, n_acc, d), jnp.float32),
        mesh=mesh,
        scratch_types=[
            pltpu.VMEM((iters, KE), jnp.int32),
            pltpu.VMEM((iters, KE), jnp.int32),
            pltpu.VMEM((nbuf, KE, d), jnp.float32),
            pltpu.VMEM_SHARED((n_acc, d), jnp.float32),
        ] + [pltpu.SemaphoreType.DMA] * nbuf,
    )


def _dinv(dp):
    return 1.0 / jnp.sqrt(dp[0, :, 0:1] + dp[1, :, 0:1] + 1.0)


def _mm(mode, a_list, s_list, gbe_list, deg_p, w, dout, n):
    cin = len(a_list)
    nblk = n // BROWS
    dcs = [128] * (dout // 128) + ([dout % 128] if dout % 128 else [])

    in_specs = [pl.BlockSpec((2, BROWS, 128), lambda i: (0, i, 0))]
    args = [deg_p]
    for c in range(cin):
        in_specs.append(pl.BlockSpec((BROWS, 128), lambda i: (i, 0)))
        args.append(a_list[c])
        if mode != "id":
            in_specs.append(pl.BlockSpec((8, 128), lambda i: (0, 0)))
            args.append(s_list[c])
            in_specs.append(pl.BlockSpec((8, 128), lambda i: (0, 0)))
            args.append(gbe_list[c])
    in_specs.append(pl.BlockSpec(w.shape, lambda i: (0, 0)))
    args.append(w)
    out_specs = [pl.BlockSpec((BROWS, dc), lambda i: (i, 0)) for dc in dcs]
    out_shape = [jax.ShapeDtypeStruct((n, dc), jnp.float32) for dc in dcs]
    inv_n = 1.0 / n

    def body(*refs):
        dp = refs[0][...]
        dinv = _dinv(dp)
        k = 1
        us = []
        for _ in range(cin):
            a = refs[k][...]
            k += 1
            if mode == "id":
                u = a
            else:
                s = refs[k][...]
                k += 1
                gbe = refs[k][...]
                k += 1
                m = s[0] * inv_n
                var = s[1] * inv_n - m * m
                istd = 1.0 / jnp.sqrt(var + EPS)
                if mode == "mu":
                    u = jnp.maximum((a - m) * istd * gbe[0] + gbe[1], 0.0)
                else:
                    u = (jnp.maximum(a, 0.0) - m) * istd * gbe[0] + gbe[1]
            us.append(u)
        wv = refs[k][...]
        k += 1
        z = jnp.dot(us[0], wv[0:128], preferred_element_type=jnp.float32)
        for c in range(1, cin):
            z = z + jnp.dot(us[c], wv[128 * c:128 * (c + 1)],
                            preferred_element_type=jnp.float32)
        z = z * dinv
        off = 0
        for o, dc in zip(refs[k:], dcs):
            o[...] = z[:, off:off + dc]
            off += dc

    outs = pl.pallas_call(body, grid=(nblk,), in_specs=in_specs,
                          out_specs=out_specs, out_shape=out_shape)(*args)
    return outs


def _combine(mode, z, p, deg_p, bpk, n, p_col_off=0):
    d = z.shape[1]
    dp_full = p.shape[2]
    nblk = n // BROWS
    stats = mode != "none"
    out_shape = [jax.ShapeDtypeStruct((n, d), jnp.float32)]
    out_specs = [pl.BlockSpec((BROWS, d), lambda i: (i, 0))]
    if stats:
        out_shape.append(jax.ShapeDtypeStruct((8, d), jnp.float32))
        out_specs.append(pl.BlockSpec((8, d), lambda i: (0, 0)))

    def body(z_ref, p_ref, dp_ref, b_ref, a_ref, *s_refs):
        i = pl.program_id(0)
        dinv = _dinv(dp_ref[...])
        pv = p_ref[...][:, :, p_col_off:p_col_off + d]
        a = dinv * (pv[0] + pv[1] + z_ref[...]) + b_ref[...][0]
        a_ref[...] = a
        if stats:
            s_ref = s_refs[0]
            t = a if mode == "mu" else jnp.maximum(a, 0.0)
            upd = jnp.concatenate(
                [jnp.sum(t, axis=0, keepdims=True),
                 jnp.sum(t * t, axis=0, keepdims=True),
                 jnp.zeros((6, d), jnp.float32)], axis=0)

            @pl.when(i == 0)
            def _():
                s_ref[...] = upd

            @pl.when(i != 0)
            def _():
                s_ref[...] = s_ref[...] + upd

    in_specs = [
        pl.BlockSpec((BROWS, d), lambda i: (i, 0)),
        pl.BlockSpec((2, BROWS, dp_full), lambda i: (0, i, 0)),
        pl.BlockSpec((2, BROWS, 128), lambda i: (0, i, 0)),
        pl.BlockSpec((8, d), lambda i: (0, 0)),
    ]
    res = pl.pallas_call(body, grid=(nblk,), in_specs=in_specs,
                         out_specs=out_specs, out_shape=out_shape)(z, p, deg_p, bpk)
    return res if stats else res[0]


def _pk8(v):
    return jnp.zeros((8, v.shape[0]), jnp.float32).at[0].set(v)


def _gbe(g, be):
    return jnp.zeros((8, g.shape[0]), jnp.float32).at[0].set(g).at[1].set(be)


def kernel(x, edge_index,
           mu_W1, mu_b1, mu_W2, mu_b2, mu_W3, mu_b3, mu_g1, mu_be1, mu_g2, mu_be2,
           log_W1, log_b1, log_W2, log_b2, log_W3, log_b3, log_g1, log_be1, log_g2, log_be2):
    n = x.shape[0]
    e = edge_index.shape[1]
    src = edge_index[0].astype(jnp.int32)
    dst = edge_index[1].astype(jnp.int32)

    per = NW * KE
    e_pad = ((e + per - 1) // per) * per
    iters = e_pad // per
    pad = e_pad - e
    src3 = jnp.concatenate([src, jnp.zeros((pad,), jnp.int32)]).reshape(NW, iters, KE)
    dst3 = jnp.concatenate([dst, jnp.full((pad,), n, jnp.int32)]).reshape(NW, iters, KE)

    n_acc = ((n + 1 + 127) // 128) * 128
    spmm128 = _make_spmm(n_acc, 128, iters)
    zeros128 = jnp.zeros((n_acc, 128), jnp.float32)

    deg_p = spmm128(jnp.ones((n, 128), jnp.float32), src3, dst3, zeros128)

    w1 = jnp.concatenate([mu_W1, log_W1], axis=1)
    z_mu1, z_log1 = _mm("id", [x], None, None, deg_p, w1, 256, n)
    p_mu1 = spmm128(z_mu1, src3, dst3, zeros128)
    a_mu1, s_mu1 = _combine("mu", z_mu1, p_mu1, deg_p, _pk8(mu_b1), n)
    p_log1 = spmm128(z_log1, src3, dst3, zeros128)
    a_log1, s_log1 = _combine("log", z_log1, p_log1, deg_p, _pk8(log_b1), n)

    z_mu2a, z_mu2b = _mm("mu", [a_mu1], [s_mu1], [_gbe(mu_g1, mu_be1)],
                         deg_p, mu_W2, 256, n)
    z_log2a, z_log2b = _mm("log", [a_log1], [s_log1], [_gbe(log_g1, log_be1)],
                           deg_p, log_W2, 256, n)
    p_mu2a = spmm128(z_mu2a, src3, dst3, zeros128)
    a_mu2a, s_mu2a = _combine("mu", z_mu2a, p_mu2a, deg_p, _pk8(mu_b2[:128]), n)
    p_mu2b = spmm128(z_mu2b, src3, dst3, zeros128)
    a_mu2b, s_mu2b = _combine("mu", z_mu2b, p_mu2b, deg_p, _pk8(mu_b2[128:]), n)
    p_log2a = spmm128(z_log2a, src3, dst3, zeros128)
    a_log2a, s_log2a = _combine("log", z_log2a, p_log2a, deg_p, _pk8(log_b2[:128]), n)
    p_log2b = spmm128(z_log2b, src3, dst3, zeros128)
    a_log2b, s_log2b = _combine("log", z_log2b, p_log2b, deg_p, _pk8(log_b2[128:]), n)

    (z_mu3,) = _mm("mu", [a_mu2a, a_mu2b], [s_mu2a, s_mu2b],
                   [_gbe(mu_g2[:128], mu_be2[:128]), _gbe(mu_g2[128:], mu_be2[128:])],
                   deg_p, mu_W3, 64, n)
    (z_log3,) = _mm("log", [a_log2a, a_log2b], [s_log2a, s_log2b],
                    [_gbe(log_g2[:128], log_be2[:128]), _gbe(log_g2[128:], log_be2[128:])],
                    deg_p, log_W3, 64, n)
    z3 = jnp.concatenate([z_mu3, z_log3], axis=1)
    p3 = spmm128(z3, src3, dst3, zeros128)
    mu = _combine("none", z_mu3, p3, deg_p, _pk8(mu_b3), n, p_col_off=0)
    log = _combine("none", z_log3, p3, deg_p, _pk8(log_b3), n, p_col_off=64)
    return (mu, log)


# speedup vs baseline: 12.3818x; 1.7559x over previous
import jax
import jax.numpy as jnp
from jax import lax
from jax.experimental import pallas as pl
from jax.experimental.pallas import tpu as pltpu
from jax.experimental.pallas import tpu_sc as plsc

EPS = 1e-5
NC = 2
NS = 16
NW = NC * NS
KE = 128
BROWS = 1000


def _make_spmm(n_acc, d, iters):
    rpt = n_acc // NS
    nbuf = 4 if iters % 4 == 0 else (2 if iters % 2 == 0 else 1)
    mesh = plsc.VectorSubcoreMesh(core_axis_name="c", subcore_axis_name="s")

    def body(zp, src3, dst3, zeros, out, src_v, dst_v, rows_v, acc, *sems):
        cid = lax.axis_index("c")
        sid = lax.axis_index("s")
        wid = cid * NS + sid
        pltpu.sync_copy(zeros.at[pl.ds(sid * rpt, rpt)], acc.at[pl.ds(sid * rpt, rpt)])
        pltpu.sync_copy(src3.at[wid], src_v)
        pltpu.sync_copy(dst3.at[wid], dst_v)
        plsc.subcore_barrier()

        for b in range(nbuf):
            pltpu.async_copy(zp.at[src_v.at[b]], rows_v.at[b], sems[b])

        def step(i2, c):
            for b in range(nbuf):
                i = i2 * nbuf + b
                pltpu.make_async_copy(zp.at[src_v.at[i]], rows_v.at[b],
                                      sems[b]).wait()
                pltpu.sync_copy(rows_v.at[b], acc.at[dst_v.at[i]], add=True)
                nxt = i + nbuf

                @pl.when(nxt < iters)
                def _():
                    pltpu.async_copy(zp.at[src_v.at[nxt]], rows_v.at[b], sems[b])
            return c

        lax.fori_loop(0, iters // nbuf, step, 0)
        plsc.subcore_barrier()
        pltpu.sync_copy(acc.at[pl.ds(sid * rpt, rpt)],
                        out.at[cid, pl.ds(sid * rpt, rpt)])

    return pl.kernel(
        body,
        out_type=jax.ShapeDtypeStruct((NC, n_acc, d), jnp.float32),
        mesh=mesh,
        scratch_types=[
            pltpu.VMEM((iters, KE), jnp.int32),
            pltpu.VMEM((iters, KE), jnp.int32),
            pltpu.VMEM((nbuf, KE, d), jnp.float32),
            pltpu.VMEM_SHARED((n_acc, d), jnp.float32),
        ] + [pltpu.SemaphoreType.DMA] * nbuf,
    )


def _dinv(dp):
    return 1.0 / jnp.sqrt(dp[0, :, 0:1] + dp[1, :, 0:1] + 1.0)


def _mm(mode, a_list, s_list, gbe_list, deg_p, w, dout, n):
    cin = len(a_list)
    nblk = n // BROWS
    dcs = [128] * (dout // 128) + ([dout % 128] if dout % 128 else [])

    in_specs = [pl.BlockSpec((2, BROWS, 128), lambda i: (0, i, 0))]
    args = [deg_p]
    for c in range(cin):
        in_specs.append(pl.BlockSpec((BROWS, 128), lambda i: (i, 0)))
        args.append(a_list[c])
        if mode != "id":
            in_specs.append(pl.BlockSpec((8, 128), lambda i: (0, 0)))
            args.append(s_list[c])
            in_specs.append(pl.BlockSpec((8, 128), lambda i: (0, 0)))
            args.append(gbe_list[c])
    in_specs.append(pl.BlockSpec(w.shape, lambda i: (0, 0)))
    args.append(w)
    out_specs = [pl.BlockSpec((BROWS, dc), lambda i: (i, 0)) for dc in dcs]
    out_shape = [jax.ShapeDtypeStruct((n, dc), jnp.float32) for dc in dcs]
    inv_n = 1.0 / n

    def body(*refs):
        dp = refs[0][...]
        dinv = _dinv(dp)
        k = 1
        us = []
        for _ in range(cin):
            a = refs[k][...]
            k += 1
            if mode == "id":
                u = a
            else:
                s = refs[k][...]
                k += 1
                gbe = refs[k][...]
                k += 1
                m = s[0] * inv_n
                var = s[1] * inv_n - m * m
                istd = 1.0 / jnp.sqrt(var + EPS)
                if mode == "mu":
                    u = jnp.maximum((a - m) * istd * gbe[0] + gbe[1], 0.0)
                else:
                    u = (jnp.maximum(a, 0.0) - m) * istd * gbe[0] + gbe[1]
            us.append(u)
        wv = refs[k][...]
        k += 1
        z = jnp.dot(us[0], wv[0:128], preferred_element_type=jnp.float32)
        for c in range(1, cin):
            z = z + jnp.dot(us[c], wv[128 * c:128 * (c + 1)],
                            preferred_element_type=jnp.float32)
        z = z * dinv
        off = 0
        for o, dc in zip(refs[k:], dcs):
            o[...] = z[:, off:off + dc]
            off += dc

    outs = pl.pallas_call(body, grid=(nblk,), in_specs=in_specs,
                          out_specs=out_specs, out_shape=out_shape)(*args)
    return outs


def _combine(mode, z, p, deg_p, bpk, n, p_col_off=0):
    d = z.shape[1]
    dp_full = p.shape[2]
    nblk = n // BROWS
    stats = mode != "none"
    out_shape = [jax.ShapeDtypeStruct((n, d), jnp.float32)]
    out_specs = [pl.BlockSpec((BROWS, d), lambda i: (i, 0))]
    if stats:
        out_shape.append(jax.ShapeDtypeStruct((8, d), jnp.float32))
        out_specs.append(pl.BlockSpec((8, d), lambda i: (0, 0)))

    def body(z_ref, p_ref, dp_ref, b_ref, a_ref, *s_refs):
        i = pl.program_id(0)
        dinv = _dinv(dp_ref[...])
        pv = p_ref[...][:, :, p_col_off:p_col_off + d]
        a = dinv * (pv[0] + pv[1] + z_ref[...]) + b_ref[...][0]
        a_ref[...] = a
        if stats:
            s_ref = s_refs[0]
            t = a if mode == "mu" else jnp.maximum(a, 0.0)
            upd = jnp.concatenate(
                [jnp.sum(t, axis=0, keepdims=True),
                 jnp.sum(t * t, axis=0, keepdims=True),
                 jnp.zeros((6, d), jnp.float32)], axis=0)

            @pl.when(i == 0)
            def _():
                s_ref[...] = upd

            @pl.when(i != 0)
            def _():
                s_ref[...] = s_ref[...] + upd

    in_specs = [
        pl.BlockSpec((BROWS, d), lambda i: (i, 0)),
        pl.BlockSpec((2, BROWS, dp_full), lambda i: (0, i, 0)),
        pl.BlockSpec((2, BROWS, 128), lambda i: (0, i, 0)),
        pl.BlockSpec((8, d), lambda i: (0, 0)),
    ]
    res = pl.pallas_call(body, grid=(nblk,), in_specs=in_specs,
                         out_specs=out_specs, out_shape=out_shape)(z, p, deg_p, bpk)
    return res if stats else res[0]


def _pk8(v):
    return jnp.zeros((8, v.shape[0]), jnp.float32).at[0].set(v)


def _gbe(g, be):
    return jnp.zeros((8, g.shape[0]), jnp.float32).at[0].set(g).at[1].set(be)


def kernel(x, edge_index,
           mu_W1, mu_b1, mu_W2, mu_b2, mu_W3, mu_b3, mu_g1, mu_be1, mu_g2, mu_be2,
           log_W1, log_b1, log_W2, log_b2, log_W3, log_b3, log_g1, log_be1, log_g2, log_be2):
    n = x.shape[0]
    e = edge_index.shape[1]
    src = edge_index[0].astype(jnp.int32)
    dst = edge_index[1].astype(jnp.int32)

    per = NW * KE
    e_pad = ((e + per - 1) // per) * per
    iters = e_pad // per
    pad = e_pad - e
    n_acc = ((n + 1 + 127) // 128) * 128
    pad_i = jnp.arange(pad, dtype=jnp.int32)
    src3 = jnp.concatenate([src, pad_i % n]).reshape(NW, iters, KE)
    dst3 = jnp.concatenate([dst, n + pad_i % (n_acc - n)]).reshape(NW, iters, KE)

    spmm128 = _make_spmm(n_acc, 128, iters)
    zeros128 = jnp.zeros((n_acc, 128), jnp.float32)

    deg_p = spmm128(jnp.ones((n, 128), jnp.float32), src3, dst3, zeros128)

    w1 = jnp.concatenate([mu_W1, log_W1], axis=1)
    z_mu1, z_log1 = _mm("id", [x], None, None, deg_p, w1, 256, n)
    p_mu1 = spmm128(z_mu1, src3, dst3, zeros128)
    a_mu1, s_mu1 = _combine("mu", z_mu1, p_mu1, deg_p, _pk8(mu_b1), n)
    p_log1 = spmm128(z_log1, src3, dst3, zeros128)
    a_log1, s_log1 = _combine("log", z_log1, p_log1, deg_p, _pk8(log_b1), n)

    z_mu2a, z_mu2b = _mm("mu", [a_mu1], [s_mu1], [_gbe(mu_g1, mu_be1)],
                         deg_p, mu_W2, 256, n)
    z_log2a, z_log2b = _mm("log", [a_log1], [s_log1], [_gbe(log_g1, log_be1)],
                           deg_p, log_W2, 256, n)
    p_mu2a = spmm128(z_mu2a, src3, dst3, zeros128)
    a_mu2a, s_mu2a = _combine("mu", z_mu2a, p_mu2a, deg_p, _pk8(mu_b2[:128]), n)
    p_mu2b = spmm128(z_mu2b, src3, dst3, zeros128)
    a_mu2b, s_mu2b = _combine("mu", z_mu2b, p_mu2b, deg_p, _pk8(mu_b2[128:]), n)
    p_log2a = spmm128(z_log2a, src3, dst3, zeros128)
    a_log2a, s_log2a = _combine("log", z_log2a, p_log2a, deg_p, _pk8(log_b2[:128]), n)
    p_log2b = spmm128(z_log2b, src3, dst3, zeros128)
    a_log2b, s_log2b = _combine("log", z_log2b, p_log2b, deg_p, _pk8(log_b2[128:]), n)

    (z_mu3,) = _mm("mu", [a_mu2a, a_mu2b], [s_mu2a, s_mu2b],
                   [_gbe(mu_g2[:128], mu_be2[:128]), _gbe(mu_g2[128:], mu_be2[128:])],
                   deg_p, mu_W3, 64, n)
    (z_log3,) = _mm("log", [a_log2a, a_log2b], [s_log2a, s_log2b],
                    [_gbe(log_g2[:128], log_be2[:128]), _gbe(log_g2[128:], log_be2[128:])],
                    deg_p, log_W3, 64, n)
    z3 = jnp.concatenate([z_mu3, z_log3], axis=1)
    p3 = spmm128(z3, src3, dst3, zeros128)
    mu = _combine("none", z_mu3, p3, deg_p, _pk8(mu_b3), n, p_col_off=0)
    log = _combine("none", z_log3, p3, deg_p, _pk8(log_b3), n, p_col_off=64)
    return (mu, log)


# speedup vs baseline: 13.6822x; 1.1050x over previous
import jax
import jax.numpy as jnp
from jax import lax
from jax.experimental import pallas as pl
from jax.experimental.pallas import tpu as pltpu
from jax.experimental.pallas import tpu_sc as plsc

EPS = 1e-5
NC = 2
NS = 16
NW = NC * NS
KE = 128
BROWS = 1000


def _make_spmm(n_acc, d, iters):
    rpt = n_acc // NS
    nbuf = 4 if iters % 4 == 0 else (2 if iters % 2 == 0 else 1)
    mesh = plsc.VectorSubcoreMesh(core_axis_name="c", subcore_axis_name="s")

    def body(zp, src3, dst3, zeros, out, src_v, dst_v, rows_v, acc, *sems):
        cid = lax.axis_index("c")
        sid = lax.axis_index("s")
        wid = cid * NS + sid
        pltpu.sync_copy(zeros.at[pl.ds(sid * rpt, rpt)], acc.at[pl.ds(sid * rpt, rpt)])
        pltpu.sync_copy(src3.at[wid], src_v)
        pltpu.sync_copy(dst3.at[wid], dst_v)
        plsc.subcore_barrier()

        for b in range(nbuf):
            pltpu.async_copy(zp.at[src_v.at[b]], rows_v.at[b], sems[b])

        def step(i2, c):
            for b in range(nbuf):
                i = i2 * nbuf + b
                pltpu.make_async_copy(zp.at[src_v.at[i]], rows_v.at[b],
                                      sems[b]).wait()
                pltpu.sync_copy(rows_v.at[b], acc.at[dst_v.at[i]], add=True)
                nxt = i + nbuf

                @pl.when(nxt < iters)
                def _():
                    pltpu.async_copy(zp.at[src_v.at[nxt]], rows_v.at[b], sems[b])
            return c

        lax.fori_loop(0, iters // nbuf, step, 0)
        plsc.subcore_barrier()
        pltpu.sync_copy(acc.at[pl.ds(sid * rpt, rpt)],
                        out.at[cid, pl.ds(sid * rpt, rpt)])

    return pl.kernel(
        body,
        out_type=jax.ShapeDtypeStruct((NC, n_acc, d), jnp.float32),
        mesh=mesh,
        scratch_types=[
            pltpu.VMEM((iters, KE), jnp.int32),
            pltpu.VMEM((iters, KE), jnp.int32),
            pltpu.VMEM((nbuf, KE, d), jnp.float32),
            pltpu.VMEM_SHARED((n_acc, d), jnp.float32),
        ] + [pltpu.SemaphoreType.DMA] * nbuf,
    )


def _make_deg(n_acc, iters):
    dd = 16
    rpt = n_acc // NS
    mesh = plsc.VectorSubcoreMesh(core_axis_name="c", subcore_axis_name="s")

    def body(ones, dst3, zeros, out, ones_v, dst_v, acc):
        cid = lax.axis_index("c")
        sid = lax.axis_index("s")
        wid = cid * NS + sid
        pltpu.sync_copy(zeros.at[pl.ds(sid * rpt, rpt)], acc.at[pl.ds(sid * rpt, rpt)])
        pltpu.sync_copy(dst3.at[wid], dst_v)
        pltpu.sync_copy(ones, ones_v)
        plsc.subcore_barrier()

        def step(i, c):
            pltpu.sync_copy(ones_v, acc.at[dst_v.at[i]], add=True)
            return c

        lax.fori_loop(0, iters, step, 0)
        plsc.subcore_barrier()
        pltpu.sync_copy(acc.at[pl.ds(sid * rpt, rpt)],
                        out.at[cid, pl.ds(sid * rpt, rpt)])

    return pl.kernel(
        body,
        out_type=jax.ShapeDtypeStruct((NC, n_acc, dd), jnp.float32),
        mesh=mesh,
        scratch_types=[
            pltpu.VMEM((KE, dd), jnp.float32),
            pltpu.VMEM((iters, KE), jnp.int32),
            pltpu.VMEM_SHARED((n_acc, dd), jnp.float32),
        ],
    )


def _dinv(dp):
    return 1.0 / jnp.sqrt(dp[0, :, 0:1] + dp[1, :, 0:1] + 1.0)


def _mm(mode, a_list, s_list, gbe_list, deg_p, w, dout, n):
    cin = len(a_list)
    nblk = n // BROWS
    dcs = [128] * (dout // 128) + ([dout % 128] if dout % 128 else [])

    in_specs = [pl.BlockSpec((2, BROWS, 16), lambda i: (0, i, 0))]
    args = [deg_p]
    for c in range(cin):
        in_specs.append(pl.BlockSpec((BROWS, 128), lambda i: (i, 0)))
        args.append(a_list[c])
        if mode != "id":
            in_specs.append(pl.BlockSpec((8, 128), lambda i: (0, 0)))
            args.append(s_list[c])
            in_specs.append(pl.BlockSpec((8, 128), lambda i: (0, 0)))
            args.append(gbe_list[c])
    in_specs.append(pl.BlockSpec(w.shape, lambda i: (0, 0)))
    args.append(w)
    out_specs = [pl.BlockSpec((BROWS, dc), lambda i: (i, 0)) for dc in dcs]
    out_shape = [jax.ShapeDtypeStruct((n, dc), jnp.float32) for dc in dcs]
    inv_n = 1.0 / n

    def body(*refs):
        dp = refs[0][...]
        dinv = _dinv(dp)
        k = 1
        us = []
        for _ in range(cin):
            a = refs[k][...]
            k += 1
            if mode == "id":
                u = a
            else:
                s = refs[k][...]
                k += 1
                gbe = refs[k][...]
                k += 1
                m = s[0] * inv_n
                var = s[1] * inv_n - m * m
                istd = 1.0 / jnp.sqrt(var + EPS)
                if mode == "mu":
                    u = jnp.maximum((a - m) * istd * gbe[0] + gbe[1], 0.0)
                else:
                    u = (jnp.maximum(a, 0.0) - m) * istd * gbe[0] + gbe[1]
            us.append(u)
        wv = refs[k][...]
        k += 1
        z = jnp.dot(us[0], wv[0:128], preferred_element_type=jnp.float32)
        for c in range(1, cin):
            z = z + jnp.dot(us[c], wv[128 * c:128 * (c + 1)],
                            preferred_element_type=jnp.float32)
        z = z * dinv
        off = 0
        for o, dc in zip(refs[k:], dcs):
            o[...] = z[:, off:off + dc]
            off += dc

    outs = pl.pallas_call(body, grid=(nblk,), in_specs=in_specs,
                          out_specs=out_specs, out_shape=out_shape)(*args)
    return outs


def _combine(mode, z, p, deg_p, bpk, n, p_col_off=0):
    d = z.shape[1]
    dp_full = p.shape[2]
    nblk = n // BROWS
    stats = mode != "none"
    out_shape = [jax.ShapeDtypeStruct((n, d), jnp.float32)]
    out_specs = [pl.BlockSpec((BROWS, d), lambda i: (i, 0))]
    if stats:
        out_shape.append(jax.ShapeDtypeStruct((8, d), jnp.float32))
        out_specs.append(pl.BlockSpec((8, d), lambda i: (0, 0)))

    def body(z_ref, p_ref, dp_ref, b_ref, a_ref, *s_refs):
        i = pl.program_id(0)
        dinv = _dinv(dp_ref[...])
        pv = p_ref[...][:, :, p_col_off:p_col_off + d]
        a = dinv * (pv[0] + pv[1] + z_ref[...]) + b_ref[...][0]
        a_ref[...] = a
        if stats:
            s_ref = s_refs[0]
            t = a if mode == "mu" else jnp.maximum(a, 0.0)
            upd = jnp.concatenate(
                [jnp.sum(t, axis=0, keepdims=True),
                 jnp.sum(t * t, axis=0, keepdims=True),
                 jnp.zeros((6, d), jnp.float32)], axis=0)

            @pl.when(i == 0)
            def _():
                s_ref[...] = upd

            @pl.when(i != 0)
            def _():
                s_ref[...] = s_ref[...] + upd

    in_specs = [
        pl.BlockSpec((BROWS, d), lambda i: (i, 0)),
        pl.BlockSpec((2, BROWS, dp_full), lambda i: (0, i, 0)),
        pl.BlockSpec((2, BROWS, 16), lambda i: (0, i, 0)),
        pl.BlockSpec((8, d), lambda i: (0, 0)),
    ]
    res = pl.pallas_call(body, grid=(nblk,), in_specs=in_specs,
                         out_specs=out_specs, out_shape=out_shape)(z, p, deg_p, bpk)
    return res if stats else res[0]


def _pk8(v):
    return jnp.zeros((8, v.shape[0]), jnp.float32).at[0].set(v)


def _gbe(g, be):
    return jnp.zeros((8, g.shape[0]), jnp.float32).at[0].set(g).at[1].set(be)


def kernel(x, edge_index,
           mu_W1, mu_b1, mu_W2, mu_b2, mu_W3, mu_b3, mu_g1, mu_be1, mu_g2, mu_be2,
           log_W1, log_b1, log_W2, log_b2, log_W3, log_b3, log_g1, log_be1, log_g2, log_be2):
    n = x.shape[0]
    e = edge_index.shape[1]
    src = edge_index[0].astype(jnp.int32)
    dst = edge_index[1].astype(jnp.int32)

    per = NW * KE
    e_pad = ((e + per - 1) // per) * per
    iters = e_pad // per
    pad = e_pad - e
    n_acc = ((n + 1 + 127) // 128) * 128
    pad_i = jnp.arange(pad, dtype=jnp.int32)
    src3 = jnp.concatenate([src, pad_i % n]).reshape(NW, iters, KE)
    dst3 = jnp.concatenate([dst, n + pad_i % (n_acc - n)]).reshape(NW, iters, KE)

    spmm128 = _make_spmm(n_acc, 128, iters)
    zeros128 = jnp.zeros((n_acc, 128), jnp.float32)

    deg_p = _make_deg(n_acc, iters)(
        jnp.ones((KE, 16), jnp.float32), dst3,
        jnp.zeros((n_acc, 16), jnp.float32))

    w1 = jnp.concatenate([mu_W1, log_W1], axis=1)
    z_mu1, z_log1 = _mm("id", [x], None, None, deg_p, w1, 256, n)
    p_mu1 = spmm128(z_mu1, src3, dst3, zeros128)
    a_mu1, s_mu1 = _combine("mu", z_mu1, p_mu1, deg_p, _pk8(mu_b1), n)
    p_log1 = spmm128(z_log1, src3, dst3, zeros128)
    a_log1, s_log1 = _combine("log", z_log1, p_log1, deg_p, _pk8(log_b1), n)

    z_mu2a, z_mu2b = _mm("mu", [a_mu1], [s_mu1], [_gbe(mu_g1, mu_be1)],
                         deg_p, mu_W2, 256, n)
    z_log2a, z_log2b = _mm("log", [a_log1], [s_log1], [_gbe(log_g1, log_be1)],
                           deg_p, log_W2, 256, n)
    p_mu2a = spmm128(z_mu2a, src3, dst3, zeros128)
    a_mu2a, s_mu2a = _combine("mu", z_mu2a, p_mu2a, deg_p, _pk8(mu_b2[:128]), n)
    p_mu2b = spmm128(z_mu2b, src3, dst3, zeros128)
    a_mu2b, s_mu2b = _combine("mu", z_mu2b, p_mu2b, deg_p, _pk8(mu_b2[128:]), n)
    p_log2a = spmm128(z_log2a, src3, dst3, zeros128)
    a_log2a, s_log2a = _combine("log", z_log2a, p_log2a, deg_p, _pk8(log_b2[:128]), n)
    p_log2b = spmm128(z_log2b, src3, dst3, zeros128)
    a_log2b, s_log2b = _combine("log", z_log2b, p_log2b, deg_p, _pk8(log_b2[128:]), n)

    (z_mu3,) = _mm("mu", [a_mu2a, a_mu2b], [s_mu2a, s_mu2b],
                   [_gbe(mu_g2[:128], mu_be2[:128]), _gbe(mu_g2[128:], mu_be2[128:])],
                   deg_p, mu_W3, 64, n)
    (z_log3,) = _mm("log", [a_log2a, a_log2b], [s_log2a, s_log2b],
                    [_gbe(log_g2[:128], log_be2[:128]), _gbe(log_g2[128:], log_be2[128:])],
                    deg_p, log_W3, 64, n)
    z3 = jnp.concatenate([z_mu3, z_log3], axis=1)
    p3 = spmm128(z3, src3, dst3, zeros128)
    mu = _combine("none", z_mu3, p3, deg_p, _pk8(mu_b3), n, p_col_off=0)
    log = _combine("none", z_log3, p3, deg_p, _pk8(log_b3), n, p_col_off=64)
    return (mu, log)


# speedup vs baseline: 13.7178x; 1.0026x over previous
import jax
import jax.numpy as jnp
from jax import lax
from jax.experimental import pallas as pl
from jax.experimental.pallas import tpu as pltpu
from jax.experimental.pallas import tpu_sc as plsc

EPS = 1e-5
NC = 2
NS = 16
NW = NC * NS
KE = 128
BROWS = 1000


def _make_spmm(n_acc, d, iters):
    rpt = n_acc // NS
    nbuf = 4 if iters % 4 == 0 else (2 if iters % 2 == 0 else 1)
    mesh = plsc.VectorSubcoreMesh(core_axis_name="c", subcore_axis_name="s")

    def body(zp, src3, dst3, zeros, out, src_v, dst_v, rows_v, acc, *sems):
        cid = lax.axis_index("c")
        sid = lax.axis_index("s")
        wid = cid * NS + sid
        pltpu.sync_copy(zeros.at[pl.ds(sid * rpt, rpt)], acc.at[pl.ds(sid * rpt, rpt)])
        pltpu.sync_copy(src3.at[wid], src_v)
        pltpu.sync_copy(dst3.at[wid], dst_v)
        plsc.subcore_barrier()

        for b in range(nbuf):
            pltpu.async_copy(zp.at[src_v.at[b]], rows_v.at[b], sems[b])

        def step(i2, c):
            for b in range(nbuf):
                i = i2 * nbuf + b
                pltpu.make_async_copy(zp.at[src_v.at[i]], rows_v.at[b],
                                      sems[b]).wait()
                pltpu.sync_copy(rows_v.at[b], acc.at[dst_v.at[i]], add=True)
                nxt = i + nbuf

                @pl.when(nxt < iters)
                def _():
                    pltpu.async_copy(zp.at[src_v.at[nxt]], rows_v.at[b], sems[b])
            return c

        lax.fori_loop(0, iters // nbuf, step, 0)
        plsc.subcore_barrier()
        pltpu.sync_copy(acc.at[pl.ds(sid * rpt, rpt)],
                        out.at[cid, pl.ds(sid * rpt, rpt)])

    return pl.kernel(
        body,
        out_type=jax.ShapeDtypeStruct((NC, n_acc, d), jnp.float32),
        mesh=mesh,
        scratch_types=[
            pltpu.VMEM((iters, KE), jnp.int32),
            pltpu.VMEM((iters, KE), jnp.int32),
            pltpu.VMEM((nbuf, KE, d), jnp.float32),
            pltpu.VMEM_SHARED((n_acc, d), jnp.float32),
        ] + [pltpu.SemaphoreType.DMA] * nbuf,
    )


def _make_spmm_multi(n_acc, iters, nchunks, d=128):
    rpt = n_acc // NS
    nbuf = 4 if iters % 4 == 0 else (2 if iters % 2 == 0 else 1)
    mesh = plsc.VectorSubcoreMesh(core_axis_name="c", subcore_axis_name="s")

    def body(*refs):
        zps = refs[:nchunks]
        src3, dst3, zeros = refs[nchunks:nchunks + 3]
        outs = refs[nchunks + 3:2 * nchunks + 3]
        src_v, dst_v, rows_v, acc = refs[2 * nchunks + 3:2 * nchunks + 7]
        sems = refs[2 * nchunks + 7:]
        cid = lax.axis_index("c")
        sid = lax.axis_index("s")
        wid = cid * NS + sid
        pltpu.sync_copy(src3.at[wid], src_v)
        pltpu.sync_copy(dst3.at[wid], dst_v)

        for c in range(nchunks):
            zp = zps[c]
            pltpu.sync_copy(zeros.at[pl.ds(sid * rpt, rpt)],
                            acc.at[pl.ds(sid * rpt, rpt)])
            plsc.subcore_barrier()

            for b in range(nbuf):
                pltpu.async_copy(zp.at[src_v.at[b]], rows_v.at[b], sems[b])

            def step(i2, cc):
                for b in range(nbuf):
                    i = i2 * nbuf + b
                    pltpu.make_async_copy(zp.at[src_v.at[i]], rows_v.at[b],
                                          sems[b]).wait()
                    pltpu.sync_copy(rows_v.at[b], acc.at[dst_v.at[i]], add=True)
                    nxt = i + nbuf

                    @pl.when(nxt < iters)
                    def _():
                        pltpu.async_copy(zp.at[src_v.at[nxt]], rows_v.at[b],
                                         sems[b])
                return cc

            lax.fori_loop(0, iters // nbuf, step, 0)
            plsc.subcore_barrier()
            pltpu.sync_copy(acc.at[pl.ds(sid * rpt, rpt)],
                            outs[c].at[cid, pl.ds(sid * rpt, rpt)])

    return pl.kernel(
        body,
        out_type=[jax.ShapeDtypeStruct((NC, n_acc, d), jnp.float32)] * nchunks,
        mesh=mesh,
        scratch_types=[
            pltpu.VMEM((iters, KE), jnp.int32),
            pltpu.VMEM((iters, KE), jnp.int32),
            pltpu.VMEM((nbuf, KE, d), jnp.float32),
            pltpu.VMEM_SHARED((n_acc, d), jnp.float32),
        ] + [pltpu.SemaphoreType.DMA] * nbuf,
    )


def _make_deg(n_acc, iters):
    dd = 16
    rpt = n_acc // NS
    mesh = plsc.VectorSubcoreMesh(core_axis_name="c", subcore_axis_name="s")

    def body(ones, dst3, zeros, out, ones_v, dst_v, acc):
        cid = lax.axis_index("c")
        sid = lax.axis_index("s")
        wid = cid * NS + sid
        pltpu.sync_copy(zeros.at[pl.ds(sid * rpt, rpt)], acc.at[pl.ds(sid * rpt, rpt)])
        pltpu.sync_copy(dst3.at[wid], dst_v)
        pltpu.sync_copy(ones, ones_v)
        plsc.subcore_barrier()

        def step(i, c):
            pltpu.sync_copy(ones_v, acc.at[dst_v.at[i]], add=True)
            return c

        lax.fori_loop(0, iters, step, 0)
        plsc.subcore_barrier()
        pltpu.sync_copy(acc.at[pl.ds(sid * rpt, rpt)],
                        out.at[cid, pl.ds(sid * rpt, rpt)])

    return pl.kernel(
        body,
        out_type=jax.ShapeDtypeStruct((NC, n_acc, dd), jnp.float32),
        mesh=mesh,
        scratch_types=[
            pltpu.VMEM((KE, dd), jnp.float32),
            pltpu.VMEM((iters, KE), jnp.int32),
            pltpu.VMEM_SHARED((n_acc, dd), jnp.float32),
        ],
    )


def _dinv(dp):
    return 1.0 / jnp.sqrt(dp[0, :, 0:1] + dp[1, :, 0:1] + 1.0)


def _mm(mode, a_list, s_list, gbe_list, deg_p, w, dout, n):
    cin = len(a_list)
    cw = a_list[0].shape[1]
    nblk = n // BROWS
    dcs = [128] * (dout // 128) + ([dout % 128] if dout % 128 else [])

    in_specs = [pl.BlockSpec((2, BROWS, 16), lambda i: (0, i, 0))]
    args = [deg_p]
    for c in range(cin):
        in_specs.append(pl.BlockSpec((BROWS, cw), lambda i: (i, 0)))
        args.append(a_list[c])
        if mode != "id":
            in_specs.append(pl.BlockSpec((8, cw), lambda i: (0, 0)))
            args.append(s_list[c])
            in_specs.append(pl.BlockSpec((8, cw), lambda i: (0, 0)))
            args.append(gbe_list[c])
    in_specs.append(pl.BlockSpec(w.shape, lambda i: (0, 0)))
    args.append(w)
    out_specs = [pl.BlockSpec((BROWS, dc), lambda i: (i, 0)) for dc in dcs]
    out_shape = [jax.ShapeDtypeStruct((n, dc), jnp.float32) for dc in dcs]
    inv_n = 1.0 / n

    def body(*refs):
        dp = refs[0][...]
        dinv = _dinv(dp)
        k = 1
        us = []
        for _ in range(cin):
            a = refs[k][...]
            k += 1
            if mode == "id":
                u = a
            else:
                s = refs[k][...]
                k += 1
                gbe = refs[k][...]
                k += 1
                m = s[0] * inv_n
                var = s[1] * inv_n - m * m
                istd = 1.0 / jnp.sqrt(var + EPS)
                if mode == "mu":
                    u = jnp.maximum((a - m) * istd * gbe[0] + gbe[1], 0.0)
                else:
                    u = (jnp.maximum(a, 0.0) - m) * istd * gbe[0] + gbe[1]
            us.append(u)
        wv = refs[k][...]
        k += 1
        z = jnp.dot(us[0], wv[0:cw], preferred_element_type=jnp.float32)
        for c in range(1, cin):
            z = z + jnp.dot(us[c], wv[cw * c:cw * (c + 1)],
                            preferred_element_type=jnp.float32)
        z = z * dinv
        off = 0
        for o, dc in zip(refs[k:], dcs):
            o[...] = z[:, off:off + dc]
            off += dc

    outs = pl.pallas_call(body, grid=(nblk,), in_specs=in_specs,
                          out_specs=out_specs, out_shape=out_shape)(*args)
    return outs


def _combine(mode, z, p, deg_p, bpk, n, p_col_off=0):
    d = z.shape[1]
    dp_full = p.shape[2]
    nblk = n // BROWS
    stats = mode != "none"
    out_shape = [jax.ShapeDtypeStruct((n, d), jnp.float32)]
    out_specs = [pl.BlockSpec((BROWS, d), lambda i: (i, 0))]
    if stats:
        out_shape.append(jax.ShapeDtypeStruct((8, d), jnp.float32))
        out_specs.append(pl.BlockSpec((8, d), lambda i: (0, 0)))

    def body(z_ref, p_ref, dp_ref, b_ref, a_ref, *s_refs):
        i = pl.program_id(0)
        dinv = _dinv(dp_ref[...])
        pv = p_ref[...][:, :, p_col_off:p_col_off + d]
        a = dinv * (pv[0] + pv[1] + z_ref[...]) + b_ref[...][0]
        a_ref[...] = a
        if stats:
            s_ref = s_refs[0]
            t = a if mode == "mu" else jnp.maximum(a, 0.0)
            upd = jnp.concatenate(
                [jnp.sum(t, axis=0, keepdims=True),
                 jnp.sum(t * t, axis=0, keepdims=True),
                 jnp.zeros((6, d), jnp.float32)], axis=0)

            @pl.when(i == 0)
            def _():
                s_ref[...] = upd

            @pl.when(i != 0)
            def _():
                s_ref[...] = s_ref[...] + upd

    in_specs = [
        pl.BlockSpec((BROWS, d), lambda i: (i, 0)),
        pl.BlockSpec((2, BROWS, dp_full), lambda i: (0, i, 0)),
        pl.BlockSpec((2, BROWS, 16), lambda i: (0, i, 0)),
        pl.BlockSpec((8, d), lambda i: (0, 0)),
    ]
    res = pl.pallas_call(body, grid=(nblk,), in_specs=in_specs,
                         out_specs=out_specs, out_shape=out_shape)(z, p, deg_p, bpk)
    return res if stats else res[0]


def _pk8(v):
    return jnp.zeros((8, v.shape[0]), jnp.float32).at[0].set(v)


def _gbe(g, be):
    return jnp.zeros((8, g.shape[0]), jnp.float32).at[0].set(g).at[1].set(be)


def kernel(x, edge_index,
           mu_W1, mu_b1, mu_W2, mu_b2, mu_W3, mu_b3, mu_g1, mu_be1, mu_g2, mu_be2,
           log_W1, log_b1, log_W2, log_b2, log_W3, log_b3, log_g1, log_be1, log_g2, log_be2):
    n = x.shape[0]
    e = edge_index.shape[1]
    src = edge_index[0].astype(jnp.int32)
    dst = edge_index[1].astype(jnp.int32)

    per = NW * KE
    e_pad = ((e + per - 1) // per) * per
    iters = e_pad // per
    pad = e_pad - e
    n_acc = ((n + 1 + 127) // 128) * 128
    pad_i = jnp.arange(pad, dtype=jnp.int32)
    src3 = jnp.concatenate([src, pad_i % n]).reshape(NW, iters, KE)
    dst3 = jnp.concatenate([dst, n + pad_i % (n_acc - n)]).reshape(NW, iters, KE)

    zeros128 = jnp.zeros((n_acc, 128), jnp.float32)

    deg_p = _make_deg(n_acc, iters)(
        jnp.ones((KE, 16), jnp.float32), dst3,
        jnp.zeros((n_acc, 16), jnp.float32))

    w1 = jnp.concatenate([mu_W1, log_W1], axis=1)
    z_mu1, z_log1 = _mm("id", [x], None, None, deg_p, w1, 256, n)
    spmm1 = _make_spmm_multi(n_acc, iters, 1)
    (p_mu1,) = spmm1(z_mu1, src3, dst3, zeros128)
    (p_log1,) = spmm1(z_log1, src3, dst3, zeros128)
    a_mu1, s_mu1 = _combine("mu", z_mu1, p_mu1, deg_p, _pk8(mu_b1), n)
    a_log1, s_log1 = _combine("log", z_log1, p_log1, deg_p, _pk8(log_b1), n)

    z2_mu = _mm("mu", [a_mu1], [s_mu1], [_gbe(mu_g1, mu_be1)],
                deg_p, mu_W2, 256, n)
    z2_log = _mm("log", [a_log1], [s_log1], [_gbe(log_g1, log_be1)],
                 deg_p, log_W2, 256, n)
    z2 = list(z2_mu) + list(z2_log)
    p2 = [spmm1(zc, src3, dst3, zeros128)[0] for zc in z2]
    bs2 = [mu_b2[:128], mu_b2[128:], log_b2[:128], log_b2[128:]]
    a2, s2 = [], []
    for c in range(4):
        ac, sc = _combine("mu" if c < 2 else "log", z2[c], p2[c], deg_p,
                          _pk8(bs2[c]), n)
        a2.append(ac)
        s2.append(sc)

    g2s = [_gbe(mu_g2[:128], mu_be2[:128]), _gbe(mu_g2[128:], mu_be2[128:])]
    g2sl = [_gbe(log_g2[:128], log_be2[:128]), _gbe(log_g2[128:], log_be2[128:])]
    (z_mu3,) = _mm("mu", a2[:2], s2[:2], g2s, deg_p, mu_W3, 64, n)
    (z_log3,) = _mm("log", a2[2:], s2[2:], g2sl, deg_p, log_W3, 64, n)
    z3 = jnp.concatenate([z_mu3, z_log3], axis=1)
    (p3,) = spmm1(z3, src3, dst3, zeros128)
    mu = _combine("none", z_mu3, p3, deg_p, _pk8(mu_b3), n, p_col_off=0)
    log = _combine("none", z_log3, p3, deg_p, _pk8(log_b3), n, p_col_off=64)
    return (mu, log)
